# baseline (speedup 1.0000x reference)
import functools

import jax
import jax.numpy as jnp
from jax import lax
from jax.experimental import pallas as pl
from jax.experimental.pallas import tpu as pltpu
from jax.experimental.pallas import tpu_sc as plsc

N = 10000
DEG = 32
D_IN = 128
D_OUT = 128

NW = 32
N_SC = 4000
S_TC = N - N_SC
CH = 8
TCH = N_SC // CH
BN = 400
BP = 2000
F = D_IN // 16


def _grn_tc_body(neigh_ref, att_ref, w_ref, b_ref, out_ref):
    neigh = neigh_ref[...]
    att = att_ref[...]
    agg = jnp.sum(neigh * att[:, :, None], axis=1)
    proj = jax.lax.dot_general(
        agg, w_ref[...], (((1,), (1,)), ((), ())),
        preferred_element_type=jnp.float32)
    out = proj + b_ref[...]
    out_ref[...] = jnp.where(out > 0, out, jnp.exp(jnp.minimum(out, 0.0)) - 1.0)


def _proj_tc_body(full_ref, agg_ref, w_ref, b_ref, out_ref):
    del full_ref
    proj = jax.lax.dot_general(
        agg_ref[...], w_ref[...], (((1,), (1,)), ((), ())),
        preferred_element_type=jnp.float32)
    out = proj + b_ref[...]
    out_ref[...] = jnp.where(out > 0, out, jnp.exp(jnp.minimum(out, 0.0)) - 1.0)


def _lane_bcast(v, d):
    idx = jnp.full((16, 1), d, jnp.int32)
    dn = lax.GatherDimensionNumbers(
        offset_dims=(), collapsed_slice_dims=(0,), start_index_map=(0,))
    return lax.gather(v, idx, dn, (1,),
                      mode=lax.GatherScatterMode.PROMISE_IN_BOUNDS)


def _sc_agg_kernel(neigh_hbm, att_hbm, out_hbm,
                   nb0, nb1, ab0, ab1, ob, semi0, semi1):
    wid = lax.axis_index("s") * 2 + lax.axis_index("c")
    base_cnt = TCH // NW
    rem = TCH % NW
    my_cnt = base_cnt + jnp.where(wid < rem, 1, 0)
    first = wid * base_cnt + jnp.minimum(wid, rem)

    def start_in(c, nbuf, abuf, sem):
        g0 = (first + c) * CH
        pltpu.async_copy(neigh_hbm.at[pl.ds(g0, CH)], nbuf, sem)
        pltpu.async_copy(att_hbm.at[pl.ds(g0, CH)], abuf, sem)

    def wait_in(c, nbuf, abuf, sem):
        g0 = (first + c) * CH
        pltpu.make_async_copy(neigh_hbm.at[pl.ds(g0, CH)], nbuf, sem).wait()
        pltpu.make_async_copy(att_hbm.at[pl.ds(g0, CH)], abuf, sem).wait()

    def compute_store(c, nbuf, abuf):
        def node_body(i, _):
            att_lo = abuf[i, pl.ds(0, 16)]
            att_hi = abuf[i, pl.ds(16, 16)]

            def d_body(dd, accs):
                av = jnp.where(dd < 2, att_lo, att_hi)
                base = (dd % 2) * 8
                new = list(accs)
                for k in range(8):
                    a = _lane_bcast(av, base + k)
                    for j in range(F):
                        new[j] = new[j] + a * nbuf[i, dd * 8 + k, pl.ds(j * 16, 16)]
                return tuple(new)

            accs = lax.fori_loop(
                0, DEG // 8, d_body,
                tuple(jnp.zeros((16,), jnp.float32) for _ in range(F)))
            for j in range(F):
                ob[i, pl.ds(j * 16, 16)] = accs[j]
            return 0

        lax.fori_loop(0, CH, node_body, 0)
        pltpu.sync_copy(ob, out_hbm.at[pl.ds((first + c) * CH, CH)])

    start_in(0, nb0, ab0, semi0)

    def c2_body(c2, _):
        c0 = 2 * c2
        start_in(c0 + 1, nb1, ab1, semi1)
        wait_in(c0, nb0, ab0, semi0)
        compute_store(c0, nb0, ab0)

        @pl.when(c0 + 2 < my_cnt)
        def _():
            start_in(c0 + 2, nb0, ab0, semi0)

        wait_in(c0 + 1, nb1, ab1, semi1)
        compute_store(c0 + 1, nb1, ab1)
        return 0

    lax.fori_loop(0, my_cnt // 2, c2_body, 0)

    @pl.when(my_cnt % 2 == 1)
    def _():
        c_last = my_cnt - 1
        wait_in(c_last, nb0, ab0, semi0)
        compute_store(c_last, nb0, ab0)


_sc_agg = functools.partial(
    pl.kernel,
    mesh=plsc.VectorSubcoreMesh(core_axis_name="c", subcore_axis_name="s"),
    out_type=jax.ShapeDtypeStruct((N_SC, D_IN), jnp.float32),
    scratch_types=[
        pltpu.VMEM((CH, DEG, D_IN), jnp.float32),
        pltpu.VMEM((CH, DEG, D_IN), jnp.float32),
        pltpu.VMEM((CH, DEG), jnp.float32),
        pltpu.VMEM((CH, DEG), jnp.float32),
        pltpu.VMEM((CH, D_IN), jnp.float32),
        pltpu.SemaphoreType.DMA,
        pltpu.SemaphoreType.DMA,
    ],
)(_sc_agg_kernel)


@jax.jit
def kernel(nodes, neighbors, attention_scores, W, bias):
    del nodes
    bias2d = bias.reshape(1, D_OUT)

    agg_sc = _sc_agg(neighbors, attention_scores)

    nsb = N_SC // BN
    out_main = pl.pallas_call(
        _grn_tc_body,
        grid=(S_TC // BN,),
        in_specs=[
            pl.BlockSpec((BN, DEG, D_IN), lambda i: (i + nsb, 0, 0)),
            pl.BlockSpec((BN, DEG), lambda i: (i + nsb, 0)),
            pl.BlockSpec((D_OUT, D_IN), lambda i: (0, 0)),
            pl.BlockSpec((1, D_OUT), lambda i: (0, 0)),
        ],
        out_specs=pl.BlockSpec((BN, D_OUT), lambda i: (i + nsb, 0)),
        out_shape=jax.ShapeDtypeStruct((N, D_OUT), jnp.float32),
    )(neighbors, attention_scores, W, bias2d)

    out = pl.pallas_call(
        _proj_tc_body,
        grid=(N_SC // BP,),
        in_specs=[
            pl.BlockSpec(memory_space=pl.ANY),
            pl.BlockSpec((BP, D_IN), lambda i: (i, 0)),
            pl.BlockSpec((D_OUT, D_IN), lambda i: (0, 0)),
            pl.BlockSpec((1, D_OUT), lambda i: (0, 0)),
        ],
        out_specs=pl.BlockSpec((BP, D_OUT), lambda i: (i, 0)),
        out_shape=jax.ShapeDtypeStruct((N, D_OUT), jnp.float32),
        input_output_aliases={0: 0},
    )(out_main, agg_sc, W, bias2d)

    return out


# speedup vs baseline: 1.0038x; 1.0038x over previous
import functools

import jax
import jax.numpy as jnp
from jax import lax
from jax.experimental import pallas as pl
from jax.experimental.pallas import tpu as pltpu
from jax.experimental.pallas import tpu_sc as plsc

N = 10000
DEG = 32
D_IN = 128
D_OUT = 128

NW = 32
N_SC = 4000
S_TC = N - N_SC
CH = 8
TCH = N_SC // CH
BN = 400
BP = 2000
F = D_IN // 16


def _grn_tc_body(neigh_ref, att_ref, w_ref, b_ref, out_ref):
    neigh = neigh_ref[...]
    att = att_ref[...]
    agg = jnp.sum(neigh * att[:, :, None], axis=1)
    proj = jax.lax.dot_general(
        agg, w_ref[...], (((1,), (1,)), ((), ())),
        preferred_element_type=jnp.float32)
    out = proj + b_ref[...]
    out_ref[...] = jnp.where(out > 0, out, jnp.exp(jnp.minimum(out, 0.0)) - 1.0)


def _proj_tc_body(full_ref, agg_ref, w_ref, b_ref, out_ref):
    del full_ref
    proj = jax.lax.dot_general(
        agg_ref[...], w_ref[...], (((1,), (1,)), ((), ())),
        preferred_element_type=jnp.float32)
    out = proj + b_ref[...]
    out_ref[...] = jnp.where(out > 0, out, jnp.exp(jnp.minimum(out, 0.0)) - 1.0)


def _lane_bcast(v, d):
    idx = jnp.full((16, 1), d, jnp.int32)
    dn = lax.GatherDimensionNumbers(
        offset_dims=(), collapsed_slice_dims=(0,), start_index_map=(0,))
    return lax.gather(v, idx, dn, (1,),
                      mode=lax.GatherScatterMode.PROMISE_IN_BOUNDS)


def _sc_agg_kernel(neigh_hbm, att_hbm, out_hbm,
                   nb0, nb1, ab0, ab1, ob, semi0, semi1):
    wid = lax.axis_index("s") * 2 + lax.axis_index("c")
    base_cnt = TCH // NW
    rem = TCH % NW
    my_cnt = base_cnt + jnp.where(wid < rem, 1, 0)
    first = wid * base_cnt + jnp.minimum(wid, rem)

    def start_in(c, nbuf, abuf, sem):
        g0 = (first + c) * CH
        pltpu.async_copy(neigh_hbm.at[pl.ds(g0, CH)], nbuf, sem)
        pltpu.async_copy(att_hbm.at[pl.ds(g0, CH)], abuf, sem)

    def wait_in(c, nbuf, abuf, sem):
        g0 = (first + c) * CH
        pltpu.make_async_copy(neigh_hbm.at[pl.ds(g0, CH)], nbuf, sem).wait()
        pltpu.make_async_copy(att_hbm.at[pl.ds(g0, CH)], abuf, sem).wait()

    def compute_store(c, nbuf, abuf):
        def node_body(i2, _):
            for u in range(2):
                i = 2 * i2 + u
                att_lo = abuf[i, pl.ds(0, 16)]
                att_hi = abuf[i, pl.ds(16, 16)]
                accs = [jnp.zeros((16,), jnp.float32) for _ in range(F)]
                for d in range(DEG):
                    a = _lane_bcast(att_lo if d < 16 else att_hi, d % 16)
                    for j in range(F):
                        accs[j] = accs[j] + a * nbuf[i, d, pl.ds(j * 16, 16)]
                for j in range(F):
                    ob[i, pl.ds(j * 16, 16)] = accs[j]
            return 0

        lax.fori_loop(0, CH // 2, node_body, 0)
        pltpu.sync_copy(ob, out_hbm.at[pl.ds((first + c) * CH, CH)])

    start_in(0, nb0, ab0, semi0)

    def c2_body(c2, _):
        c0 = 2 * c2
        start_in(c0 + 1, nb1, ab1, semi1)
        wait_in(c0, nb0, ab0, semi0)
        compute_store(c0, nb0, ab0)

        @pl.when(c0 + 2 < my_cnt)
        def _():
            start_in(c0 + 2, nb0, ab0, semi0)

        wait_in(c0 + 1, nb1, ab1, semi1)
        compute_store(c0 + 1, nb1, ab1)
        return 0

    lax.fori_loop(0, my_cnt // 2, c2_body, 0)

    @pl.when(my_cnt % 2 == 1)
    def _():
        c_last = my_cnt - 1
        wait_in(c_last, nb0, ab0, semi0)
        compute_store(c_last, nb0, ab0)


_sc_agg = functools.partial(
    pl.kernel,
    mesh=plsc.VectorSubcoreMesh(core_axis_name="c", subcore_axis_name="s"),
    out_type=jax.ShapeDtypeStruct((N_SC, D_IN), jnp.float32),
    scratch_types=[
        pltpu.VMEM((CH, DEG, D_IN), jnp.float32),
        pltpu.VMEM((CH, DEG, D_IN), jnp.float32),
        pltpu.VMEM((CH, DEG), jnp.float32),
        pltpu.VMEM((CH, DEG), jnp.float32),
        pltpu.VMEM((CH, D_IN), jnp.float32),
        pltpu.SemaphoreType.DMA,
        pltpu.SemaphoreType.DMA,
    ],
)(_sc_agg_kernel)


@jax.jit
def kernel(nodes, neighbors, attention_scores, W, bias):
    del nodes
    bias2d = bias.reshape(1, D_OUT)

    agg_sc = _sc_agg(neighbors, attention_scores)

    nsb = N_SC // BN
    out_main = pl.pallas_call(
        _grn_tc_body,
        grid=(S_TC // BN,),
        in_specs=[
            pl.BlockSpec((BN, DEG, D_IN), lambda i: (i + nsb, 0, 0)),
            pl.BlockSpec((BN, DEG), lambda i: (i + nsb, 0)),
            pl.BlockSpec((D_OUT, D_IN), lambda i: (0, 0)),
            pl.BlockSpec((1, D_OUT), lambda i: (0, 0)),
        ],
        out_specs=pl.BlockSpec((BN, D_OUT), lambda i: (i + nsb, 0)),
        out_shape=jax.ShapeDtypeStruct((N, D_OUT), jnp.float32),
    )(neighbors, attention_scores, W, bias2d)

    out = pl.pallas_call(
        _proj_tc_body,
        grid=(N_SC // BP,),
        in_specs=[
            pl.BlockSpec(memory_space=pl.ANY),
            pl.BlockSpec((BP, D_IN), lambda i: (i, 0)),
            pl.BlockSpec((D_OUT, D_IN), lambda i: (0, 0)),
            pl.BlockSpec((1, D_OUT), lambda i: (0, 0)),
        ],
        out_specs=pl.BlockSpec((BP, D_OUT), lambda i: (i, 0)),
        out_shape=jax.ShapeDtypeStruct((N, D_OUT), jnp.float32),
        input_output_aliases={0: 0},
    )(out_main, agg_sc, W, bias2d)

    return out
